# baseline (speedup 1.0000x reference)
import jax
import jax.numpy as jnp
from jax import lax
from jax.experimental import pallas as pl
from jax.experimental.pallas import tpu as pltpu


def _siamese_kernel(c1_ref, c2_ref, table_ref, bias_ref,
                    out_ref, vec_ref):
    core = pl.program_id(0)
    vpg = out_ref.shape[1]
    ogroups = out_ref.shape[0]
    pairs = ogroups * vpg
    seq = c1_ref.shape[1]
    v, h = table_ref.shape

    tab = table_ref.reshape(v // 8, 8, h)

    def encode(c_ref, gslot0):
        def body(g, carry):
            rows = []
            for vloc in range(vpg):
                row = core * pairs + g * vpg + vloc

                def fetch(t):
                    cr = c_ref[row, t]
                    return pltpu.roll(tab[cr >> 3], cr & 7, axis=0)

                acc = fetch(0)
                for t in range(1, seq):
                    acc = jnp.maximum(acc, fetch(t))
                rows.append(acc[0:1, :])
            vec_ref[gslot0 + g] = jnp.concatenate(rows, axis=0)
            return carry
        lax.fori_loop(0, pairs // vpg, body, 0)

    encode(c1_ref, 0)
    encode(c2_ref, pairs // vpg)

    pooled = vec_ref[...] + bias_ref[...]
    v1 = pooled[:ogroups]
    v2 = pooled[ogroups:]
    eps2 = 1e-12 * 1e-12
    n1 = jnp.maximum(jnp.sum(v1 * v1, axis=2), eps2)
    n2 = jnp.maximum(jnp.sum(v2 * v2, axis=2), eps2)
    dt = jnp.sum(v1 * v2, axis=2)
    out_ref[...] = 1.0 - dt * lax.rsqrt(n1 * n2)


def kernel(table_fused, bias, sentence1, sentence2):
    v, h = table_fused.shape
    b, l = sentence1.shape
    n_cores = 2 if b % 16 == 0 else 1
    pairs = b // n_cores
    vpg = 2
    s1 = sentence1.astype(jnp.int32)
    s2 = sentence2.astype(jnp.int32)
    call = pl.pallas_call(
        _siamese_kernel,
        grid=(n_cores,),
        in_specs=[
            pl.BlockSpec(memory_space=pltpu.SMEM),
            pl.BlockSpec(memory_space=pltpu.SMEM),
            pl.BlockSpec(memory_space=pltpu.MemorySpace.VMEM),
            pl.BlockSpec((1, h), lambda i: (0, 0)),
        ],
        out_specs=pl.BlockSpec((pairs // vpg, vpg), lambda i: (i, 0)),
        out_shape=jax.ShapeDtypeStruct((b // vpg, vpg), jnp.float32),
        scratch_shapes=[pltpu.VMEM((2 * pairs // vpg, vpg, h), jnp.float32)],
        compiler_params=pltpu.CompilerParams(
            dimension_semantics=("parallel",),
            vmem_limit_bytes=48 * 1024 * 1024),
    )
    c1 = ((s1 >> 3) << 3) | ((8 - (s1 & 7)) & 7)
    c2 = ((s2 >> 3) << 3) | ((8 - (s2 & 7)) & 7)
    out = call(c1, c2, table_fused, bias)
    return out.reshape(-1)


# speedup vs baseline: 1.0716x; 1.0716x over previous
import jax
import jax.numpy as jnp
from jax import lax
from jax.experimental import pallas as pl
from jax.experimental.pallas import tpu as pltpu


def _siamese_kernel(c1_ref, c2_ref, table_ref, bias_ref,
                    out_ref, vec_ref):
    core = pl.program_id(0)
    vpg = out_ref.shape[1]
    ogroups = out_ref.shape[0]
    pairs = ogroups * vpg
    seq = c1_ref.shape[1]
    v, h = table_ref.shape

    tab = table_ref.reshape(v // 8, 8, h)

    def encode(c_ref, gslot0):
        def body(g, carry):
            rows = []
            for vloc in range(vpg):
                row = core * pairs + g * vpg + vloc

                def fetch(t):
                    cr = c_ref[row, t]
                    return pltpu.roll(tab[cr >> 3], cr & 7, axis=0)

                acc = fetch(0)
                for t in range(1, seq):
                    acc = jnp.maximum(acc, fetch(t))
                rows.append(acc[0:1, :])
            vec_ref[gslot0 + g] = jnp.concatenate(rows, axis=0)
            return carry
        lax.fori_loop(0, pairs // vpg, body, 0)

    encode(c1_ref, 0)
    encode(c2_ref, pairs // vpg)

    pooled = vec_ref[...] + bias_ref[...]
    v1 = pooled[:ogroups]
    v2 = pooled[ogroups:]
    eps2 = 1e-12 * 1e-12
    n1 = jnp.maximum(jnp.sum(v1 * v1, axis=2), eps2)
    n2 = jnp.maximum(jnp.sum(v2 * v2, axis=2), eps2)
    dt = jnp.sum(v1 * v2, axis=2)
    out_ref[...] = 1.0 - dt * lax.rsqrt(n1 * n2)


def kernel(table_fused, bias, sentence1, sentence2):
    v, h = table_fused.shape
    b, l = sentence1.shape
    n_cores = 2 if b % 16 == 0 else 1
    pairs = b // n_cores
    vpg = 4
    s1 = sentence1.astype(jnp.int32)
    s2 = sentence2.astype(jnp.int32)
    call = pl.pallas_call(
        _siamese_kernel,
        grid=(n_cores,),
        in_specs=[
            pl.BlockSpec(memory_space=pltpu.SMEM),
            pl.BlockSpec(memory_space=pltpu.SMEM),
            pl.BlockSpec(memory_space=pltpu.MemorySpace.VMEM),
            pl.BlockSpec((1, h), lambda i: (0, 0)),
        ],
        out_specs=pl.BlockSpec((pairs // vpg, vpg), lambda i: (i, 0)),
        out_shape=jax.ShapeDtypeStruct((b // vpg, vpg), jnp.float32),
        scratch_shapes=[pltpu.VMEM((2 * pairs // vpg, vpg, h), jnp.float32)],
        compiler_params=pltpu.CompilerParams(
            dimension_semantics=("parallel",),
            vmem_limit_bytes=48 * 1024 * 1024),
    )
    c1 = ((s1 >> 3) << 3) | ((8 - (s1 & 7)) & 7)
    c2 = ((s2 >> 3) << 3) | ((8 - (s2 & 7)) & 7)
    out = call(c1, c2, table_fused, bias)
    return out.reshape(-1)
